# baseline (speedup 1.0000x reference)
import functools

import jax
import jax.numpy as jnp
from jax import lax
from jax.experimental import pallas as pl
from jax.experimental.pallas import tpu as pltpu
from jax.experimental.pallas import tpu_sc as plsc

NC = 2
NS = 16
NW = NC * NS
CHUNK = 128
DH = 16


def _mesh():
    return plsc.VectorSubcoreMesh(core_axis_name="c", subcore_axis_name="s")


_SC_PARAMS = pltpu.CompilerParams(use_tc_tiling_on_sc=False)


def _group_k(nfull):
    for k in range(16, 0, -1):
        if nfull % k == 0:
            return k
    return 1


@functools.lru_cache(maxsize=None)
def _make_deg(E, NPAD):
    EPW = E // NW
    NFULL = EPW // CHUNK
    TAIL = EPW - NFULL * CHUNK
    RPW = NPAD // NS
    K = _group_k(NFULL)
    NG = NFULL // K

    @functools.partial(
        pl.kernel,
        out_type=jax.ShapeDtypeStruct((NC * NPAD,), jnp.float32),
        mesh=_mesh(),
        compiler_params=_SC_PARAMS,
        scratch_types=[
            pltpu.VMEM((NFULL, CHUNK), jnp.int32),
            pltpu.VMEM((CHUNK,), jnp.float32),
            pltpu.VMEM((max(TAIL, 8),), jnp.int32),
            pltpu.VMEM((max(TAIL, 8),), jnp.float32),
            pltpu.VMEM((RPW,), jnp.float32),
            pltpu.VMEM_SHARED((NPAD,), jnp.float32),
            pltpu.SemaphoreType.DMA,
            pltpu.SemaphoreType.DMA,
        ],
    )
    def deg_kernel(ei_hbm, out_hbm,
                   idx_v, ones_v, idx_t, ones_t, stage, acc, sem, semi):
        c = lax.axis_index("c")
        s = lax.axis_index("s")
        wid = c * NS + s
        r0 = s * RPW

        def pre(j, carry):
            base = wid * EPW + j * CHUNK
            pltpu.async_copy(ei_hbm.at[1, pl.ds(base, CHUNK)],
                             idx_v.at[j], semi)
            return carry

        lax.fori_loop(0, NFULL, pre, 0)
        if TAIL:
            baset = wid * EPW + NFULL * CHUNK
            pltpu.async_copy(ei_hbm.at[1, pl.ds(baset, TAIL)],
                             idx_t.at[pl.ds(0, TAIL)], semi)
        zv = jnp.zeros((16,), jnp.float32)
        ov = jnp.ones((16,), jnp.float32)
        for i in range(RPW // 16):
            stage[pl.ds(16 * i, 16)] = zv
        if RPW % 16:
            stage[pl.ds(RPW - 16, 16)] = zv
        pltpu.sync_copy(stage, acc.at[pl.ds(r0, RPW)])
        for i in range(CHUNK // 16):
            ones_v[pl.ds(16 * i, 16)] = ov
        if TAIL:
            ones_t[pl.ds(0, 16)] = ov

        def pre_drain(j, carry):
            base = wid * EPW + j * CHUNK
            pltpu.make_async_copy(ei_hbm.at[1, pl.ds(base, CHUNK)],
                                  idx_v.at[j], semi).wait()
            return carry

        lax.fori_loop(0, NFULL, pre_drain, 0)
        if TAIL:
            pltpu.make_async_copy(ei_hbm.at[1, pl.ds(baset, TAIL)],
                                  idx_t.at[pl.ds(0, TAIL)], semi).wait()
        plsc.subcore_barrier()

        def body(j, carry):
            ds = [pltpu.async_copy(ones_v, acc.at[idx_v.at[j * K + b]], sem,
                                   add=True)
                  for b in range(K)]
            for d in ds:
                d.wait()
            return carry

        lax.fori_loop(0, NG, body, 0)
        if TAIL:
            pltpu.sync_copy(ones_t.at[pl.ds(0, TAIL)],
                            acc.at[idx_t.at[pl.ds(0, TAIL)]], add=True)
        plsc.subcore_barrier()
        pltpu.sync_copy(acc.at[pl.ds(r0, RPW)], stage)
        pltpu.sync_copy(stage, out_hbm.at[pl.ds(c * NPAD + r0, RPW)])

    return deg_kernel


@functools.lru_cache(maxsize=None)
def _make_agg(E, N, NPAD):
    EPW = E // NW
    NFULL = EPW // CHUNK
    TAIL = EPW - NFULL * CHUNK
    RPW = NPAD // NS
    K = _group_k(NFULL)
    NG = NFULL // K
    D = DH

    @functools.partial(
        pl.kernel,
        out_type=jax.ShapeDtypeStruct((NC, NPAD, D), jnp.float32),
        mesh=_mesh(),
        compiler_params=_SC_PARAMS,
        scratch_types=[
            pltpu.VMEM((NFULL, CHUNK), jnp.int32),
            pltpu.VMEM((NFULL, CHUNK), jnp.int32),
            pltpu.VMEM((2, K, CHUNK, D), jnp.float32),
            pltpu.VMEM((max(TAIL, 8),), jnp.int32),
            pltpu.VMEM((max(TAIL, 8),), jnp.int32),
            pltpu.VMEM((max(TAIL, 8), D), jnp.float32),
            pltpu.VMEM((RPW, D), jnp.float32),
            pltpu.VMEM_SHARED((NPAD, D), jnp.float32),
            pltpu.SemaphoreType.DMA((2,)),
            pltpu.SemaphoreType.DMA((2,)),
            pltpu.SemaphoreType.DMA,
        ],
    )
    def agg_kernel(ei_hbm, h_hbm, out_hbm,
                   sidx, didx, rows, sidx_t, didx_t, rows_t, stage,
                   acc, semg, sems, semi):
        c = lax.axis_index("c")
        s = lax.axis_index("s")
        wid = c * NS + s
        r0 = s * RPW

        def pre(j, carry):
            base = wid * EPW + j * CHUNK
            pltpu.async_copy(ei_hbm.at[0, pl.ds(base, CHUNK)],
                             sidx.at[j], semi)
            pltpu.async_copy(ei_hbm.at[1, pl.ds(base, CHUNK)],
                             didx.at[j], semi)
            return carry

        lax.fori_loop(0, NFULL, pre, 0)
        if TAIL:
            baset = wid * EPW + NFULL * CHUNK
            pltpu.async_copy(ei_hbm.at[0, pl.ds(baset, TAIL)],
                             sidx_t.at[pl.ds(0, TAIL)], semi)
            pltpu.async_copy(ei_hbm.at[1, pl.ds(baset, TAIL)],
                             didx_t.at[pl.ds(0, TAIL)], semi)
        zv = jnp.zeros((16,), jnp.float32)

        def zero_stage(j, carry):
            stage[j, :] = zv
            return carry

        lax.fori_loop(0, RPW, zero_stage, 0)
        pltpu.sync_copy(stage, acc.at[pl.ds(r0, RPW)])

        def pre_drain(j, carry):
            base = wid * EPW + j * CHUNK
            pltpu.make_async_copy(ei_hbm.at[0, pl.ds(base, CHUNK)],
                                  sidx.at[j], semi).wait()
            pltpu.make_async_copy(ei_hbm.at[1, pl.ds(base, CHUNK)],
                                  didx.at[j], semi).wait()
            return carry

        lax.fori_loop(0, NFULL, pre_drain, 0)
        if TAIL:
            pltpu.make_async_copy(ei_hbm.at[0, pl.ds(baset, TAIL)],
                                  sidx_t.at[pl.ds(0, TAIL)], semi).wait()
            pltpu.make_async_copy(ei_hbm.at[1, pl.ds(baset, TAIL)],
                                  didx_t.at[pl.ds(0, TAIL)], semi).wait()
        plsc.subcore_barrier()

        def fire_g(g, h):
            return [pltpu.async_copy(h_hbm.at[sidx.at[g * K + b]],
                                     rows.at[h, b], semg.at[h])
                    for b in range(K)]

        def fire_s(g, h):
            return [pltpu.async_copy(rows.at[h, b],
                                     acc.at[didx.at[g * K + b]],
                                     sems.at[h], add=True)
                    for b in range(K)]

        def drain(ds):
            for d in ds:
                d.wait()

        def drain_g(h):
            for b in range(K):
                pltpu.make_async_copy(h_hbm.at[sidx.at[b]],
                                      rows.at[h, b], semg.at[h]).wait()

        if NG % 2 == 0 and NG >= 2:
            fire_g(0, 0)

            def body(p, carry):
                ga = 2 * p
                drain_g(0)
                sa = fire_s(ga, 0)
                gb = fire_g(ga + 1, 1)
                drain(sa)

                @pl.when(p < NG // 2 - 1)
                def _():
                    fire_g(ga + 2, 0)
                drain(gb)
                drain(fire_s(ga + 1, 1))
                return carry

            lax.fori_loop(0, NG // 2, body, 0)
        else:
            def body1(j, carry):
                drain(fire_g(j, 0))
                drain(fire_s(j, 0))
                return carry

            lax.fori_loop(0, NG, body1, 0)
        if TAIL:
            pltpu.async_copy(h_hbm.at[sidx_t.at[pl.ds(0, TAIL)]],
                             rows_t.at[pl.ds(0, TAIL)], semg.at[0]).wait()
            pltpu.sync_copy(rows_t.at[pl.ds(0, TAIL)],
                            acc.at[didx_t.at[pl.ds(0, TAIL)]], add=True)
        plsc.subcore_barrier()
        pltpu.sync_copy(acc.at[pl.ds(r0, RPW)], stage)
        pltpu.sync_copy(stage, out_hbm.at[c, pl.ds(r0, RPW)])

    return agg_kernel


def _tc1(x, W1, degw, S16, N, NPAD):
    NB = N // 8
    NBP = NPAD // 8

    def body(x_ref, w_ref, degw_ref, s16_ref, hs_ref, dis_ref):
        deg = degw_ref[0] + degw_ref[1] + 1.0
        dis8 = lax.rsqrt(deg)
        dis16 = jnp.dot(dis8, s16_ref[...],
                        preferred_element_type=jnp.float32)
        h16 = jnp.dot(x_ref[...], w_ref[...],
                      preferred_element_type=jnp.float32)
        hv = h16.reshape(NB, 8, 128)
        for s in range(8):
            sl = slice(DH * s, DH * (s + 1))
            hs_ref[pl.ds(0, NB), pl.ds(DH * s, DH)] = (
                hv[:, s, sl] * dis16[:NB, sl])
        hs_ref[pl.ds(NB, NBP - NB), :] = jnp.zeros(
            (NBP - NB, 128), jnp.float32)
        dis_ref[...] = dis16

    return pl.pallas_call(
        body,
        out_shape=[
            jax.ShapeDtypeStruct((NBP, 128), jnp.float32),
            jax.ShapeDtypeStruct((NBP, 128), jnp.float32),
        ],
    )(x, W1, degw, S16)


def _tc2(aggp, hs, dis16, b1t, W2K, NPAD):
    def body(aggp_ref, hs_ref, dis_ref, b1_ref, w_ref, out_ref):
        agg = aggp_ref[0] + aggp_ref[1]
        d = dis_ref[...]
        tot = (agg + hs_ref[...]) * d + b1_ref[...]
        z = jnp.maximum(tot, 0.0)
        h2 = jnp.dot(z, w_ref[...], preferred_element_type=jnp.float32)
        out_ref[...] = h2 * d

    return pl.pallas_call(
        body,
        out_shape=jax.ShapeDtypeStruct((NPAD // 8, 128), jnp.float32),
    )(aggp, hs, dis16, b1t, W2K)


def _tc3(aggp, h2s, dis16, b2t, shifts, maxb, sumb, rsel, N, NPAD, DO):
    OFF = 2.0e6
    NB = N // 8

    def body(aggp_ref, h2s_ref, dis_ref, b2_ref, sh_ref, mb_ref, sb_ref,
             rs_ref, out_ref):
        agg = aggp_ref[0] + aggp_ref[1]
        d = dis_ref[...]
        o = (agg + h2s_ref[...]) * d + b2_ref[...]
        lane = lax.broadcasted_iota(jnp.int32, o.shape, 1)
        valid = (lane & 15) < DO
        y = jnp.where(valid, o + OFF, 0.0)
        m = y
        for k in range(4):
            m = jnp.maximum(m, jnp.dot(m, sh_ref[k],
                                       preferred_element_type=jnp.float32))
        c = jnp.dot(m, mb_ref[...],
                    preferred_element_type=jnp.float32) - OFF
        e = jnp.where(valid, jnp.exp(o - c), 0.0)
        s = jnp.dot(e, sb_ref[...], preferred_element_type=jnp.float32)
        lsm = o - c - jnp.log(s)
        for q in range(8):
            um = jnp.dot(lsm[:NB], rs_ref[q],
                         preferred_element_type=jnp.float32)
            out_ref[q] = um[:, :DO]

    return pl.pallas_call(
        body,
        out_shape=jax.ShapeDtypeStruct((8, NB, DO), jnp.float32),
    )(aggp, h2s, dis16, b2t, shifts, maxb, sumb, rsel)


def kernel(x, edge_index, W1, b1, W2, b2):
    N, _ = x.shape
    DO = W2.shape[1]
    E = edge_index.shape[1]

    NPAD = -(-N // (NS * 8)) * (NS * 8)

    eye8 = jnp.eye(8, dtype=jnp.float32)
    W1r = jnp.tile(W1, (1, 8))
    W2p = jnp.zeros((DH, DH), jnp.float32).at[:, :DO].set(W2)
    W2K = jnp.kron(eye8, W2p)
    S16 = jnp.kron(eye8, jnp.ones((1, DH), jnp.float32))
    shifts = jnp.stack([
        jnp.kron(eye8, jnp.eye(DH, k=-k, dtype=jnp.float32))
        for k in (1, 2, 4, 8)])
    maxb = jnp.kron(
        eye8, jnp.zeros((DH, DH), jnp.float32).at[0, :].set(1.0))
    sumb = jnp.kron(eye8, jnp.ones((DH, DH), jnp.float32))
    rsel = jnp.stack([
        jnp.zeros((128, DH), jnp.float32).at[DH * q:DH * (q + 1), :].set(
            jnp.eye(DH, dtype=jnp.float32))
        for q in range(8)])
    b1t = jnp.tile(b1, 8).reshape(1, 128)
    b2t = jnp.tile(jnp.concatenate(
        [b2, jnp.zeros((DH - DO,), jnp.float32)]), 8).reshape(1, 128)

    degp = _make_deg(E, NPAD)(edge_index)
    degw = degp.reshape(NC, NPAD // 8, 8)
    hsP, dis16 = _tc1(x, W1r, degw, S16, N, NPAD)
    hs = hsP.reshape(NPAD, DH)
    aggp1 = _make_agg(E, N, NPAD)(edge_index, hs)
    aggp1P = aggp1.reshape(NC, NPAD // 8, 128)
    h2sP = _tc2(aggp1P, hsP, dis16, b1t, W2K, NPAD)
    h2s = h2sP.reshape(NPAD, DH)
    aggp2 = _make_agg(E, N, NPAD)(edge_index, h2s)
    aggp2P = aggp2.reshape(NC, NPAD // 8, 128)
    out3 = _tc3(aggp2P, h2sP, dis16, b2t, shifts, maxb, sumb, rsel,
                N, NPAD, DO)
    return out3.transpose(1, 0, 2).reshape(N, DO)


# speedup vs baseline: 1.0874x; 1.0874x over previous
import functools

import jax
import jax.numpy as jnp
from jax import lax
from jax.experimental import pallas as pl
from jax.experimental.pallas import tpu as pltpu
from jax.experimental.pallas import tpu_sc as plsc

NC = 2
NS = 16
NW = NC * NS
CHUNK = 128
DH = 16


def _mesh():
    return plsc.VectorSubcoreMesh(core_axis_name="c", subcore_axis_name="s")


_SC_PARAMS = pltpu.CompilerParams(use_tc_tiling_on_sc=False)


def _group_k(nfull):
    for k in range(16, 0, -1):
        if nfull % k == 0:
            return k
    return 1


@functools.lru_cache(maxsize=None)
def _make_deg(E, NPAD):
    EPW = E // NW
    NFULL = EPW // CHUNK
    TAIL = EPW - NFULL * CHUNK
    RPW = NPAD // NS
    K = _group_k(NFULL)
    NG = NFULL // K

    @functools.partial(
        pl.kernel,
        out_type=jax.ShapeDtypeStruct((NC * NPAD,), jnp.float32),
        mesh=_mesh(),
        compiler_params=_SC_PARAMS,
        scratch_types=[
            pltpu.VMEM((NFULL, CHUNK), jnp.int32),
            pltpu.VMEM((CHUNK,), jnp.float32),
            pltpu.VMEM((max(TAIL, 8),), jnp.int32),
            pltpu.VMEM((max(TAIL, 8),), jnp.float32),
            pltpu.VMEM((RPW,), jnp.float32),
            pltpu.VMEM_SHARED((NPAD,), jnp.float32),
            pltpu.SemaphoreType.DMA,
            pltpu.SemaphoreType.DMA,
        ],
    )
    def deg_kernel(ei_hbm, out_hbm,
                   idx_v, ones_v, idx_t, ones_t, stage, acc, sem, semi):
        c = lax.axis_index("c")
        s = lax.axis_index("s")
        wid = c * NS + s
        r0 = s * RPW

        def pre(j, carry):
            base = wid * EPW + j * CHUNK
            pltpu.async_copy(ei_hbm.at[1, pl.ds(base, CHUNK)],
                             idx_v.at[j], semi)
            return carry

        lax.fori_loop(0, NFULL, pre, 0)
        if TAIL:
            baset = wid * EPW + NFULL * CHUNK
            pltpu.async_copy(ei_hbm.at[1, pl.ds(baset, TAIL)],
                             idx_t.at[pl.ds(0, TAIL)], semi)
        zv = jnp.zeros((16,), jnp.float32)
        ov = jnp.ones((16,), jnp.float32)
        for i in range(RPW // 16):
            stage[pl.ds(16 * i, 16)] = zv
        if RPW % 16:
            stage[pl.ds(RPW - 16, 16)] = zv
        pltpu.sync_copy(stage, acc.at[pl.ds(r0, RPW)])
        for i in range(CHUNK // 16):
            ones_v[pl.ds(16 * i, 16)] = ov
        if TAIL:
            ones_t[pl.ds(0, 16)] = ov

        def pre_drain(j, carry):
            base = wid * EPW + j * CHUNK
            pltpu.make_async_copy(ei_hbm.at[1, pl.ds(base, CHUNK)],
                                  idx_v.at[j], semi).wait()
            return carry

        lax.fori_loop(0, NFULL, pre_drain, 0)
        if TAIL:
            pltpu.make_async_copy(ei_hbm.at[1, pl.ds(baset, TAIL)],
                                  idx_t.at[pl.ds(0, TAIL)], semi).wait()
        plsc.subcore_barrier()

        def body(j, carry):
            ds = [pltpu.async_copy(ones_v, acc.at[idx_v.at[j * K + b]], sem,
                                   add=True)
                  for b in range(K)]
            for d in ds:
                d.wait()
            return carry

        lax.fori_loop(0, NG, body, 0)
        if TAIL:
            pltpu.sync_copy(ones_t.at[pl.ds(0, TAIL)],
                            acc.at[idx_t.at[pl.ds(0, TAIL)]], add=True)
        plsc.subcore_barrier()
        pltpu.sync_copy(acc.at[pl.ds(r0, RPW)], stage)
        pltpu.sync_copy(stage, out_hbm.at[pl.ds(c * NPAD + r0, RPW)])

    return deg_kernel


@functools.lru_cache(maxsize=None)
def _make_agg(E, N, NPAD):
    EPW = E // NW
    NFULL = EPW // CHUNK
    TAIL = EPW - NFULL * CHUNK
    RPW = NPAD // NS
    K = _group_k(NFULL)
    NG = NFULL // K
    D = DH

    @functools.partial(
        pl.kernel,
        out_type=jax.ShapeDtypeStruct((NC, NPAD, D), jnp.float32),
        mesh=_mesh(),
        compiler_params=_SC_PARAMS,
        scratch_types=[
            pltpu.VMEM((NFULL, CHUNK), jnp.int32),
            pltpu.VMEM((NFULL, CHUNK), jnp.int32),
            pltpu.VMEM((2, K, CHUNK, D), jnp.float32),
            pltpu.VMEM((max(TAIL, 8),), jnp.int32),
            pltpu.VMEM((max(TAIL, 8),), jnp.int32),
            pltpu.VMEM((max(TAIL, 8), D), jnp.float32),
            pltpu.VMEM((RPW, D), jnp.float32),
            pltpu.VMEM_SHARED((NPAD, D), jnp.float32),
            pltpu.SemaphoreType.DMA((2,)),
            pltpu.SemaphoreType.DMA((2,)),
            pltpu.SemaphoreType.DMA,
        ],
    )
    def agg_kernel(ei_hbm, h_hbm, out_hbm,
                   sidx, didx, rows, sidx_t, didx_t, rows_t, stage,
                   acc, semg, sems, semi):
        c = lax.axis_index("c")
        s = lax.axis_index("s")
        wid = c * NS + s
        r0 = s * RPW

        def pre(j, carry):
            base = wid * EPW + j * CHUNK
            pltpu.async_copy(ei_hbm.at[0, pl.ds(base, CHUNK)],
                             sidx.at[j], semi)
            pltpu.async_copy(ei_hbm.at[1, pl.ds(base, CHUNK)],
                             didx.at[j], semi)
            return carry

        lax.fori_loop(0, NFULL, pre, 0)
        if TAIL:
            baset = wid * EPW + NFULL * CHUNK
            pltpu.async_copy(ei_hbm.at[0, pl.ds(baset, TAIL)],
                             sidx_t.at[pl.ds(0, TAIL)], semi)
            pltpu.async_copy(ei_hbm.at[1, pl.ds(baset, TAIL)],
                             didx_t.at[pl.ds(0, TAIL)], semi)
        zv = jnp.zeros((16,), jnp.float32)

        def zero_stage(j, carry):
            stage[j, :] = zv
            return carry

        lax.fori_loop(0, RPW, zero_stage, 0)
        pltpu.sync_copy(stage, acc.at[pl.ds(r0, RPW)])

        def pre_drain(j, carry):
            base = wid * EPW + j * CHUNK
            pltpu.make_async_copy(ei_hbm.at[0, pl.ds(base, CHUNK)],
                                  sidx.at[j], semi).wait()
            pltpu.make_async_copy(ei_hbm.at[1, pl.ds(base, CHUNK)],
                                  didx.at[j], semi).wait()
            return carry

        lax.fori_loop(0, NFULL, pre_drain, 0)
        if TAIL:
            pltpu.make_async_copy(ei_hbm.at[0, pl.ds(baset, TAIL)],
                                  sidx_t.at[pl.ds(0, TAIL)], semi).wait()
            pltpu.make_async_copy(ei_hbm.at[1, pl.ds(baset, TAIL)],
                                  didx_t.at[pl.ds(0, TAIL)], semi).wait()
        plsc.subcore_barrier()

        def fire_g(g, h):
            return [pltpu.async_copy(h_hbm.at[sidx.at[g * K + b]],
                                     rows.at[h, b], semg.at[h])
                    for b in range(K)]

        def fire_s(g, h):
            return [pltpu.async_copy(rows.at[h, b],
                                     acc.at[didx.at[g * K + b]],
                                     sems.at[h], add=True)
                    for b in range(K)]

        def drain(ds):
            for d in ds:
                d.wait()

        def drain_g(h):
            for b in range(K):
                pltpu.make_async_copy(h_hbm.at[sidx.at[b]],
                                      rows.at[h, b], semg.at[h]).wait()

        if NG % 2 == 0 and NG >= 2:
            fire_g(0, 0)

            def body(p, carry):
                ga = 2 * p
                drain_g(0)
                sa = fire_s(ga, 0)
                gb = fire_g(ga + 1, 1)
                drain(sa)

                @pl.when(p < NG // 2 - 1)
                def _():
                    fire_g(ga + 2, 0)
                drain(gb)
                drain(fire_s(ga + 1, 1))
                return carry

            lax.fori_loop(0, NG // 2, body, 0)
        else:
            def body1(j, carry):
                drain(fire_g(j, 0))
                drain(fire_s(j, 0))
                return carry

            lax.fori_loop(0, NG, body1, 0)
        if TAIL:
            pltpu.async_copy(h_hbm.at[sidx_t.at[pl.ds(0, TAIL)]],
                             rows_t.at[pl.ds(0, TAIL)], semg.at[0]).wait()
            pltpu.sync_copy(rows_t.at[pl.ds(0, TAIL)],
                            acc.at[didx_t.at[pl.ds(0, TAIL)]], add=True)
        plsc.subcore_barrier()
        pltpu.sync_copy(acc.at[pl.ds(r0, RPW)], stage)
        pltpu.sync_copy(stage, out_hbm.at[c, pl.ds(r0, RPW)])

    return agg_kernel


def _tc1(x, W1, degw, S16, N, NPAD):
    NB = N // 8
    NBP = NPAD // 8

    def body(x_ref, w_ref, degw_ref, s16_ref, hs_ref, dis_ref):
        deg = degw_ref[0] + degw_ref[1] + 1.0
        dis8 = lax.rsqrt(deg)
        dis16 = jnp.dot(dis8, s16_ref[...],
                        preferred_element_type=jnp.float32)
        h16 = jnp.dot(x_ref[...], w_ref[...],
                      preferred_element_type=jnp.float32)
        hv = h16.reshape(NB, 8, 128)
        for s in range(8):
            sl = slice(DH * s, DH * (s + 1))
            hs_ref[pl.ds(0, NB), pl.ds(DH * s, DH)] = (
                hv[:, s, sl] * dis16[:NB, sl])
        hs_ref[pl.ds(NB, NBP - NB), :] = jnp.zeros(
            (NBP - NB, 128), jnp.float32)
        dis_ref[...] = dis16

    return pl.pallas_call(
        body,
        out_shape=[
            jax.ShapeDtypeStruct((NBP, 128), jnp.float32),
            jax.ShapeDtypeStruct((NBP, 128), jnp.float32),
        ],
    )(x, W1, degw, S16)


def _tc2(aggp, hs, dis16, b1t, W2K, NPAD):
    def body(aggp_ref, hs_ref, dis_ref, b1_ref, w_ref, out_ref):
        agg = aggp_ref[0] + aggp_ref[1]
        d = dis_ref[...]
        tot = (agg + hs_ref[...]) * d + b1_ref[...]
        z = jnp.maximum(tot, 0.0)
        h2 = jnp.dot(z, w_ref[...], preferred_element_type=jnp.float32)
        out_ref[...] = h2 * d

    return pl.pallas_call(
        body,
        out_shape=jax.ShapeDtypeStruct((NPAD // 8, 128), jnp.float32),
    )(aggp, hs, dis16, b1t, W2K)


def _tc3(aggp, h2s, dis16, b2t, shifts, maxb, sumb, rsel, N, NPAD, DO):
    OFF = 2.0e6
    NB = N // 8

    def body(aggp_ref, h2s_ref, dis_ref, b2_ref, sh_ref, mb_ref, sb_ref,
             rs_ref, out_ref):
        agg = aggp_ref[0] + aggp_ref[1]
        d = dis_ref[...]
        o = (agg + h2s_ref[...]) * d + b2_ref[...]
        lane = lax.broadcasted_iota(jnp.int32, o.shape, 1)
        valid = (lane & 15) < DO
        y = jnp.where(valid, o + OFF, 0.0)
        m = y
        for k in range(4):
            m = jnp.maximum(m, jnp.dot(m, sh_ref[k],
                                       preferred_element_type=jnp.float32))
        c = jnp.dot(m, mb_ref[...],
                    preferred_element_type=jnp.float32) - OFF
        e = jnp.where(valid, jnp.exp(o - c), 0.0)
        s = jnp.dot(e, sb_ref[...], preferred_element_type=jnp.float32)
        lsm = o - c - jnp.log(s)
        for q in range(8):
            um = jnp.dot(lsm[:NB], rs_ref[q],
                         preferred_element_type=jnp.float32)
            out_ref[:, q, pl.ds(0, DH)] = um

    return pl.pallas_call(
        body,
        out_shape=jax.ShapeDtypeStruct((NB, 8, 128), jnp.float32),
    )(aggp, h2s, dis16, b2t, shifts, maxb, sumb, rsel)


def kernel(x, edge_index, W1, b1, W2, b2):
    N, _ = x.shape
    DO = W2.shape[1]
    E = edge_index.shape[1]

    NPAD = -(-N // (NS * 8)) * (NS * 8)

    eye8 = jnp.eye(8, dtype=jnp.float32)
    W1r = jnp.tile(W1, (1, 8))
    W2p = jnp.zeros((DH, DH), jnp.float32).at[:, :DO].set(W2)
    W2K = jnp.kron(eye8, W2p)
    S16 = jnp.kron(eye8, jnp.ones((1, DH), jnp.float32))
    shifts = jnp.stack([
        jnp.kron(eye8, jnp.eye(DH, k=-k, dtype=jnp.float32))
        for k in (1, 2, 4, 8)])
    maxb = jnp.kron(
        eye8, jnp.zeros((DH, DH), jnp.float32).at[0, :].set(1.0))
    sumb = jnp.kron(eye8, jnp.ones((DH, DH), jnp.float32))
    rsel = jnp.stack([
        jnp.zeros((128, DH), jnp.float32).at[DH * q:DH * (q + 1), :].set(
            jnp.eye(DH, dtype=jnp.float32))
        for q in range(8)])
    b1t = jnp.tile(b1, 8).reshape(1, 128)
    b2t = jnp.tile(jnp.concatenate(
        [b2, jnp.zeros((DH - DO,), jnp.float32)]), 8).reshape(1, 128)

    degp = _make_deg(E, NPAD)(edge_index)
    degw = degp.reshape(NC, NPAD // 8, 8)
    hsP, dis16 = _tc1(x, W1r, degw, S16, N, NPAD)
    hs = hsP.reshape(NPAD, DH)
    aggp1 = _make_agg(E, N, NPAD)(edge_index, hs)
    aggp1P = aggp1.reshape(NC, NPAD // 8, 128)
    h2sP = _tc2(aggp1P, hsP, dis16, b1t, W2K, NPAD)
    h2s = h2sP.reshape(NPAD, DH)
    aggp2 = _make_agg(E, N, NPAD)(edge_index, h2s)
    aggp2P = aggp2.reshape(NC, NPAD // 8, 128)
    out3 = _tc3(aggp2P, h2sP, dis16, b2t, shifts, maxb, sumb, rsel,
                N, NPAD, DO)
    return out3.reshape(N, 128)[:, :DO]


# speedup vs baseline: 1.1723x; 1.0781x over previous
import functools

import jax
import jax.numpy as jnp
from jax import lax
from jax.experimental import pallas as pl
from jax.experimental.pallas import tpu as pltpu
from jax.experimental.pallas import tpu_sc as plsc

NC = 2
NS = 16
NW = NC * NS
CHUNK = 128
DH = 16


def _mesh():
    return plsc.VectorSubcoreMesh(core_axis_name="c", subcore_axis_name="s")


_SC_PARAMS = pltpu.CompilerParams(use_tc_tiling_on_sc=False)


def _group_k(nfull):
    for k in range(16, 0, -1):
        if nfull % k == 0:
            return k
    return 1


@functools.lru_cache(maxsize=None)
def _make_deg(E, NPAD):
    EPW = E // NW
    NFULL = EPW // CHUNK
    TAIL = EPW - NFULL * CHUNK
    RPW = NPAD // NS
    K = _group_k(NFULL)
    NG = NFULL // K

    @functools.partial(
        pl.kernel,
        out_type=jax.ShapeDtypeStruct((NC * NPAD,), jnp.float32),
        mesh=_mesh(),
        compiler_params=_SC_PARAMS,
        scratch_types=[
            pltpu.VMEM((NFULL, CHUNK), jnp.int32),
            pltpu.VMEM((CHUNK,), jnp.float32),
            pltpu.VMEM((max(TAIL, 8),), jnp.int32),
            pltpu.VMEM((max(TAIL, 8),), jnp.float32),
            pltpu.VMEM((RPW,), jnp.float32),
            pltpu.VMEM_SHARED((NPAD,), jnp.float32),
            pltpu.SemaphoreType.DMA,
            pltpu.SemaphoreType.DMA,
        ],
    )
    def deg_kernel(ei_hbm, out_hbm,
                   idx_v, ones_v, idx_t, ones_t, stage, acc, sem, semi):
        c = lax.axis_index("c")
        s = lax.axis_index("s")
        wid = c * NS + s
        r0 = s * RPW

        def pre(j, carry):
            base = wid * EPW + j * CHUNK
            pltpu.async_copy(ei_hbm.at[1, pl.ds(base, CHUNK)],
                             idx_v.at[j], semi)
            return carry

        lax.fori_loop(0, NFULL, pre, 0)
        if TAIL:
            baset = wid * EPW + NFULL * CHUNK
            pltpu.async_copy(ei_hbm.at[1, pl.ds(baset, TAIL)],
                             idx_t.at[pl.ds(0, TAIL)], semi)
        zv = jnp.zeros((16,), jnp.float32)
        ov = jnp.ones((16,), jnp.float32)
        for i in range(RPW // 16):
            stage[pl.ds(16 * i, 16)] = zv
        if RPW % 16:
            stage[pl.ds(RPW - 16, 16)] = zv
        pltpu.sync_copy(stage, acc.at[pl.ds(r0, RPW)])
        for i in range(CHUNK // 16):
            ones_v[pl.ds(16 * i, 16)] = ov
        if TAIL:
            ones_t[pl.ds(0, 16)] = ov

        def pre_drain(j, carry):
            base = wid * EPW + j * CHUNK
            pltpu.make_async_copy(ei_hbm.at[1, pl.ds(base, CHUNK)],
                                  idx_v.at[j], semi).wait()
            return carry

        lax.fori_loop(0, NFULL, pre_drain, 0)
        if TAIL:
            pltpu.make_async_copy(ei_hbm.at[1, pl.ds(baset, TAIL)],
                                  idx_t.at[pl.ds(0, TAIL)], semi).wait()
        plsc.subcore_barrier()

        def body(j, carry):
            ds = [pltpu.async_copy(ones_v, acc.at[idx_v.at[j * K + b]], sem,
                                   add=True)
                  for b in range(K)]
            for d in ds:
                d.wait()
            return carry

        lax.fori_loop(0, NG, body, 0)
        if TAIL:
            pltpu.sync_copy(ones_t.at[pl.ds(0, TAIL)],
                            acc.at[idx_t.at[pl.ds(0, TAIL)]], add=True)
        plsc.subcore_barrier()
        pltpu.sync_copy(acc.at[pl.ds(r0, RPW)], stage)
        pltpu.sync_copy(stage, out_hbm.at[pl.ds(c * NPAD + r0, RPW)])

    return deg_kernel


@functools.lru_cache(maxsize=None)
def _make_agg(E, N, NPAD):
    EPW = E // NW
    NFULL = EPW // CHUNK
    TAIL = EPW - NFULL * CHUNK
    RPW = NPAD // NS
    K = _group_k(NFULL)
    NG = NFULL // K
    D = DH

    @functools.partial(
        pl.kernel,
        out_type=jax.ShapeDtypeStruct((NC, NPAD, D), jnp.float32),
        mesh=_mesh(),
        compiler_params=_SC_PARAMS,
        scratch_types=[
            pltpu.VMEM((NFULL, CHUNK), jnp.int32),
            pltpu.VMEM((NFULL, CHUNK), jnp.int32),
            pltpu.VMEM((2, K, CHUNK, D), jnp.float32),
            pltpu.VMEM((max(TAIL, 8),), jnp.int32),
            pltpu.VMEM((max(TAIL, 8),), jnp.int32),
            pltpu.VMEM((max(TAIL, 8), D), jnp.float32),
            pltpu.VMEM((RPW, D), jnp.float32),
            pltpu.VMEM_SHARED((NPAD, D), jnp.float32),
            pltpu.SemaphoreType.DMA((2,)),
            pltpu.SemaphoreType.DMA((2,)),
            pltpu.SemaphoreType.DMA,
        ],
    )
    def agg_kernel(ei_hbm, h_hbm, out_hbm,
                   sidx, didx, rows, sidx_t, didx_t, rows_t, stage,
                   acc, semg, sems, semi):
        c = lax.axis_index("c")
        s = lax.axis_index("s")
        wid = c * NS + s
        r0 = s * RPW

        def pre(j, carry):
            base = wid * EPW + j * CHUNK
            pltpu.async_copy(ei_hbm.at[0, pl.ds(base, CHUNK)],
                             sidx.at[j], semi)
            pltpu.async_copy(ei_hbm.at[1, pl.ds(base, CHUNK)],
                             didx.at[j], semi)
            return carry

        lax.fori_loop(0, NFULL, pre, 0)
        if TAIL:
            baset = wid * EPW + NFULL * CHUNK
            pltpu.async_copy(ei_hbm.at[0, pl.ds(baset, TAIL)],
                             sidx_t.at[pl.ds(0, TAIL)], semi)
            pltpu.async_copy(ei_hbm.at[1, pl.ds(baset, TAIL)],
                             didx_t.at[pl.ds(0, TAIL)], semi)
        zv = jnp.zeros((16,), jnp.float32)

        def zero_stage(j, carry):
            stage[j, :] = zv
            return carry

        lax.fori_loop(0, RPW, zero_stage, 0)
        pltpu.sync_copy(stage, acc.at[pl.ds(r0, RPW)])

        def pre_drain(j, carry):
            base = wid * EPW + j * CHUNK
            pltpu.make_async_copy(ei_hbm.at[0, pl.ds(base, CHUNK)],
                                  sidx.at[j], semi).wait()
            pltpu.make_async_copy(ei_hbm.at[1, pl.ds(base, CHUNK)],
                                  didx.at[j], semi).wait()
            return carry

        lax.fori_loop(0, NFULL, pre_drain, 0)
        if TAIL:
            pltpu.make_async_copy(ei_hbm.at[0, pl.ds(baset, TAIL)],
                                  sidx_t.at[pl.ds(0, TAIL)], semi).wait()
            pltpu.make_async_copy(ei_hbm.at[1, pl.ds(baset, TAIL)],
                                  didx_t.at[pl.ds(0, TAIL)], semi).wait()
        plsc.subcore_barrier()

        def fire_g(g, h):
            return [pltpu.async_copy(h_hbm.at[sidx.at[g * K + b]],
                                     rows.at[h, b], semg.at[h])
                    for b in range(K)]

        def fire_s(g, h):
            return [pltpu.async_copy(rows.at[h, b],
                                     acc.at[didx.at[g * K + b]],
                                     sems.at[h], add=True)
                    for b in range(K)]

        def drain(ds):
            for d in ds:
                d.wait()

        def fire_s_interleaved(g, h):
            out = []
            for b in range(K):
                pltpu.make_async_copy(h_hbm.at[sidx.at[b]],
                                      rows.at[h, b], semg.at[h]).wait()
                out.append(pltpu.async_copy(rows.at[h, b],
                                            acc.at[didx.at[g * K + b]],
                                            sems.at[h], add=True))
            return out

        if NG % 2 == 0 and NG >= 2:
            fire_g(0, 0)

            def body(p, carry):
                ga = 2 * p
                sa = fire_s_interleaved(ga, 0)
                gb = fire_g(ga + 1, 1)
                drain(sa)

                @pl.when(p < NG // 2 - 1)
                def _():
                    fire_g(ga + 2, 0)
                sb = fire_s_interleaved(ga + 1, 1)
                drain(sb)
                return carry

            lax.fori_loop(0, NG // 2, body, 0)
        else:
            def body1(j, carry):
                drain(fire_g(j, 0))
                drain(fire_s(j, 0))
                return carry

            lax.fori_loop(0, NG, body1, 0)
        if TAIL:
            pltpu.async_copy(h_hbm.at[sidx_t.at[pl.ds(0, TAIL)]],
                             rows_t.at[pl.ds(0, TAIL)], semg.at[0]).wait()
            pltpu.sync_copy(rows_t.at[pl.ds(0, TAIL)],
                            acc.at[didx_t.at[pl.ds(0, TAIL)]], add=True)
        plsc.subcore_barrier()
        pltpu.sync_copy(acc.at[pl.ds(r0, RPW)], stage)
        pltpu.sync_copy(stage, out_hbm.at[c, pl.ds(r0, RPW)])

    return agg_kernel


def _tc1a(x, W1r, N, NPAD):
    NB = N // 8
    NBP = NPAD // 8

    def body(x_ref, w_ref, hp_ref):
        h16 = jnp.dot(x_ref[...], w_ref[...],
                      preferred_element_type=jnp.float32)
        hv = h16.reshape(NB, 8, 128)
        for s in range(8):
            sl = slice(DH * s, DH * (s + 1))
            hp_ref[pl.ds(0, NB), pl.ds(DH * s, DH)] = hv[:, s, sl]
        hp_ref[pl.ds(NB, NBP - NB), :] = jnp.zeros(
            (NBP - NB, 128), jnp.float32)

    return pl.pallas_call(
        body,
        out_shape=jax.ShapeDtypeStruct((NBP, 128), jnp.float32),
    )(x, W1r)


def _tc1b(hp, degw, S16, NPAD):
    def body(hp_ref, degw_ref, s16_ref, hs_ref, dis_ref):
        deg = degw_ref[0] + degw_ref[1] + 1.0
        dis8 = lax.rsqrt(deg)
        dis16 = jnp.dot(dis8, s16_ref[...],
                        preferred_element_type=jnp.float32)
        hs_ref[...] = hp_ref[...] * dis16
        dis_ref[...] = dis16

    return pl.pallas_call(
        body,
        out_shape=[
            jax.ShapeDtypeStruct((NPAD // 8, 128), jnp.float32),
            jax.ShapeDtypeStruct((NPAD // 8, 128), jnp.float32),
        ],
    )(hp, degw, S16)


def _tc2(aggp, hs, dis16, b1t, W2K, NPAD):
    def body(aggp_ref, hs_ref, dis_ref, b1_ref, w_ref, out_ref):
        agg = aggp_ref[0] + aggp_ref[1]
        d = dis_ref[...]
        tot = (agg + hs_ref[...]) * d + b1_ref[...]
        z = jnp.maximum(tot, 0.0)
        h2 = jnp.dot(z, w_ref[...], preferred_element_type=jnp.float32)
        out_ref[...] = h2 * d

    return pl.pallas_call(
        body,
        out_shape=jax.ShapeDtypeStruct((NPAD // 8, 128), jnp.float32),
    )(aggp, hs, dis16, b1t, W2K)


def _tc3(aggp, h2s, dis16, b2t, shifts, maxb, sumb, rsel, N, NPAD, DO):
    OFF = 2.0e6
    NB = N // 8

    def body(aggp_ref, h2s_ref, dis_ref, b2_ref, sh_ref, mb_ref, sb_ref,
             rs_ref, out_ref):
        agg = aggp_ref[0] + aggp_ref[1]
        d = dis_ref[...]
        o = (agg + h2s_ref[...]) * d + b2_ref[...]
        lane = lax.broadcasted_iota(jnp.int32, o.shape, 1)
        valid = (lane & 15) < DO
        y = jnp.where(valid, o + OFF, 0.0)
        m = y
        for k in range(4):
            m = jnp.maximum(m, jnp.dot(m, sh_ref[k],
                                       preferred_element_type=jnp.float32))
        c = jnp.dot(m, mb_ref[...],
                    preferred_element_type=jnp.float32) - OFF
        e = jnp.where(valid, jnp.exp(o - c), 0.0)
        s = jnp.dot(e, sb_ref[...], preferred_element_type=jnp.float32)
        lsm = o - c - jnp.log(s)
        for q in range(8):
            um = jnp.dot(lsm[:NB], rs_ref[q],
                         preferred_element_type=jnp.float32)
            out_ref[:, q, pl.ds(0, DH)] = um

    return pl.pallas_call(
        body,
        out_shape=jax.ShapeDtypeStruct((NB, 8, 128), jnp.float32),
    )(aggp, h2s, dis16, b2t, shifts, maxb, sumb, rsel)


def kernel(x, edge_index, W1, b1, W2, b2):
    N, _ = x.shape
    DO = W2.shape[1]
    E = edge_index.shape[1]

    NPAD = -(-N // (NS * 8)) * (NS * 8)

    eye8 = jnp.eye(8, dtype=jnp.float32)
    W1r = jnp.tile(W1, (1, 8))
    W2p = jnp.zeros((DH, DH), jnp.float32).at[:, :DO].set(W2)
    W2K = jnp.kron(eye8, W2p)
    S16 = jnp.kron(eye8, jnp.ones((1, DH), jnp.float32))
    shifts = jnp.stack([
        jnp.kron(eye8, jnp.eye(DH, k=-k, dtype=jnp.float32))
        for k in (1, 2, 4, 8)])
    maxb = jnp.kron(
        eye8, jnp.zeros((DH, DH), jnp.float32).at[0, :].set(1.0))
    sumb = jnp.kron(eye8, jnp.ones((DH, DH), jnp.float32))
    rsel = jnp.stack([
        jnp.zeros((128, DH), jnp.float32).at[DH * q:DH * (q + 1), :].set(
            jnp.eye(DH, dtype=jnp.float32))
        for q in range(8)])
    b1t = jnp.tile(b1, 8).reshape(1, 128)
    b2t = jnp.tile(jnp.concatenate(
        [b2, jnp.zeros((DH - DO,), jnp.float32)]), 8).reshape(1, 128)

    degp = _make_deg(E, NPAD)(edge_index)
    degw = degp.reshape(NC, NPAD // 8, 8)
    hpP = _tc1a(x, W1r, N, NPAD)
    hsP, dis16 = _tc1b(hpP, degw, S16, NPAD)
    hs = hsP.reshape(NPAD, DH)
    aggp1 = _make_agg(E, N, NPAD)(edge_index, hs)
    aggp1P = aggp1.reshape(NC, NPAD // 8, 128)
    h2sP = _tc2(aggp1P, hsP, dis16, b1t, W2K, NPAD)
    h2s = h2sP.reshape(NPAD, DH)
    aggp2 = _make_agg(E, N, NPAD)(edge_index, h2s)
    aggp2P = aggp2.reshape(NC, NPAD // 8, 128)
    out3 = _tc3(aggp2P, h2sP, dis16, b2t, shifts, maxb, sumb, rsel,
                N, NPAD, DO)
    return out3.reshape(N, 128)[:, :DO]
